# baseline (device time: 85087 ns/iter reference)
import jax
import jax.numpy as jnp
from jax import lax
from jax.experimental import pallas as pl
from jax.experimental.pallas import tpu as pltpu

N_DEV = 4


def kernel(partial, resid, gamma):
    x = partial.reshape(partial.shape[-2], partial.shape[-1])
    m, n = x.shape
    gamma2d = gamma.reshape(1, n)

    def body(x_ref, resid_ref, gamma_ref, out_ref, comm_ref, send_sems, recv_sems):
        my = lax.axis_index("i")
        left = lax.rem(my + (N_DEV - 1), N_DEV)
        right = lax.rem(my + 1, N_DEV)

        barrier_sem = pltpu.get_barrier_semaphore()
        for nbr in (left, right):
            pl.semaphore_signal(
                barrier_sem, inc=1,
                device_id=(nbr,), device_id_type=pl.DeviceIdType.MESH,
            )
        pl.semaphore_wait(barrier_sem, 2)

        comm_ref[0] = x_ref[...].astype(jnp.bfloat16)
        out_ref[...] = x_ref[...]

        for h in range(N_DEV - 1):
            rdma = pltpu.make_async_remote_copy(
                src_ref=comm_ref.at[h],
                dst_ref=comm_ref.at[h + 1],
                send_sem=send_sems.at[h],
                recv_sem=recv_sems.at[h],
                device_id=(right,),
                device_id_type=pl.DeviceIdType.MESH,
            )
            rdma.start()
            rdma.wait()
            out_ref[...] += comm_ref[h + 1].astype(jnp.float32)

        y = out_ref[...] + resid_ref[...]
        ms = jnp.mean(y * y, axis=-1, keepdims=True)
        out_ref[...] = y * lax.rsqrt(ms + 1e-6) * gamma_ref[...]

    return pl.pallas_call(
        body,
        out_shape=jax.ShapeDtypeStruct((m, n), jnp.float32),
        in_specs=[
            pl.BlockSpec(memory_space=pltpu.VMEM),
            pl.BlockSpec(memory_space=pltpu.VMEM),
            pl.BlockSpec(memory_space=pltpu.VMEM),
        ],
        out_specs=pl.BlockSpec(memory_space=pltpu.VMEM),
        scratch_shapes=[
            pltpu.VMEM((N_DEV, m, n), jnp.bfloat16),
            pltpu.SemaphoreType.DMA((N_DEV - 1,)),
            pltpu.SemaphoreType.DMA((N_DEV - 1,)),
        ],
        compiler_params=pltpu.CompilerParams(collective_id=0),
    )(x, resid, gamma2d)


# device time: 33952 ns/iter; 2.5061x vs baseline; 2.5061x over previous
import jax
import jax.numpy as jnp
from jax import lax
from jax.experimental import pallas as pl
from jax.experimental.pallas import tpu as pltpu

N_DEV = 4


def kernel(partial, resid, gamma):
    x = partial.reshape(partial.shape[-2], partial.shape[-1])
    m, n = x.shape
    bs = m // 8
    gamma2d = gamma.reshape(1, n)

    def body(x_ref, resid_ref, gamma_ref, out_ref,
             xb_ref, r1_ref, r2_ref, ag_ref, send_sems, recv_sems):
        my = lax.axis_index("i")
        p1 = my ^ 1
        p2 = 3 - my

        def rowA(j):
            return j * bs

        def rowB(j):
            return (4 + j) * bs

        barrier_sem = pltpu.get_barrier_semaphore()
        for nbr in (p1, p2):
            pl.semaphore_signal(
                barrier_sem, inc=1,
                device_id=(nbr,), device_id_type=pl.DeviceIdType.MESH,
            )
        pl.semaphore_wait(barrier_sem, 2)

        xb_ref[...] = x_ref[...].astype(jnp.bfloat16)

        def make(src_ref, dst_ref, off, partner, i):
            return pltpu.make_async_remote_copy(
                src_ref=src_ref.at[pl.ds(off, bs), :],
                dst_ref=dst_ref.at[pl.ds(off, bs), :],
                send_sem=send_sems.at[i],
                recv_sem=recv_sems.at[i],
                device_id=(partner,),
                device_id_type=pl.DeviceIdType.MESH,
            )

        def start_stage(transfers):
            rdmas = [make(*t) for t in transfers]
            for r in rdmas:
                r.start()
            return rdmas

        def wait_stage(rdmas):
            for r in rdmas:
                r.wait()

        s1 = start_stage([
            (xb_ref, r1_ref, rowA(p1), p1, 0),
            (xb_ref, r1_ref, rowA(p2 ^ 1), p1, 1),
            (xb_ref, r1_ref, rowB(p2), p2, 2),
            (xb_ref, r1_ref, rowB(p2 ^ 1), p2, 3),
        ])
        wait_stage(s1)

        for off in (rowA(my), rowA(p2), rowB(my), rowB(p1)):
            r1_ref[pl.ds(off, bs), :] = (
                r1_ref[pl.ds(off, bs), :] + xb_ref[pl.ds(off, bs), :]
            )

        s2 = start_stage([
            (r1_ref, r2_ref, rowA(p2), p2, 4),
            (r1_ref, r2_ref, rowB(p1), p1, 5),
        ])
        wait_stage(s2)

        for off in (rowA(my), rowB(my)):
            s = r1_ref[pl.ds(off, bs), :] + r2_ref[pl.ds(off, bs), :]
            y = s.astype(jnp.float32) + resid_ref[pl.ds(off, bs), :]
            ms = jnp.mean(y * y, axis=-1, keepdims=True)
            o = y * lax.rsqrt(ms + 1e-6) * gamma_ref[...]
            out_ref[pl.ds(off, bs), :] = o
            ag_ref[pl.ds(off, bs), :] = o.astype(jnp.bfloat16)

        s3 = start_stage([
            (ag_ref, ag_ref, rowA(my), p2, 6),
            (ag_ref, ag_ref, rowB(my), p1, 7),
        ])
        wait_stage(s3)

        s4 = start_stage([
            (ag_ref, ag_ref, rowA(my), p1, 8),
            (ag_ref, ag_ref, rowA(p2), p1, 9),
            (ag_ref, ag_ref, rowB(my), p2, 10),
            (ag_ref, ag_ref, rowB(p1), p2, 11),
        ])
        for off in (rowA(p2), rowB(p1)):
            out_ref[pl.ds(off, bs), :] = (
                ag_ref[pl.ds(off, bs), :].astype(jnp.float32)
            )
        wait_stage(s4)
        for off in (rowA(p1), rowA(p2 ^ 1), rowB(p2), rowB(p2 ^ 1)):
            out_ref[pl.ds(off, bs), :] = (
                ag_ref[pl.ds(off, bs), :].astype(jnp.float32)
            )

    return pl.pallas_call(
        body,
        out_shape=jax.ShapeDtypeStruct((m, n), jnp.float32),
        in_specs=[
            pl.BlockSpec(memory_space=pltpu.VMEM),
            pl.BlockSpec(memory_space=pltpu.VMEM),
            pl.BlockSpec(memory_space=pltpu.VMEM),
        ],
        out_specs=pl.BlockSpec(memory_space=pltpu.VMEM),
        scratch_shapes=[
            pltpu.VMEM((m, n), jnp.bfloat16),
            pltpu.VMEM((m, n), jnp.bfloat16),
            pltpu.VMEM((m, n), jnp.bfloat16),
            pltpu.VMEM((m, n), jnp.bfloat16),
            pltpu.SemaphoreType.DMA((12,)),
            pltpu.SemaphoreType.DMA((12,)),
        ],
        compiler_params=pltpu.CompilerParams(collective_id=0),
    )(x, resid, gamma2d)


# device time: 32641 ns/iter; 2.6068x vs baseline; 1.0402x over previous
import jax
import jax.numpy as jnp
from jax import lax
from jax.experimental import pallas as pl
from jax.experimental.pallas import tpu as pltpu

N_DEV = 4


def kernel(partial, resid, gamma):
    x = partial.reshape(partial.shape[-2], partial.shape[-1])
    m, n = x.shape
    bs = m // 8
    gamma2d = gamma.reshape(1, n)

    def body(x_ref, resid_ref, gamma_ref, out_ref,
             xb_ref, r1_ref, r2_ref, ag_ref, send_sems, recv_sems):
        my = lax.axis_index("i")
        p1 = my ^ 1
        p2 = 3 - my

        def rowA(j):
            return j * bs

        def rowB(j):
            return (4 + j) * bs

        barrier_sem = pltpu.get_barrier_semaphore()
        for nbr in (p1, p2):
            pl.semaphore_signal(
                barrier_sem, inc=1,
                device_id=(nbr,), device_id_type=pl.DeviceIdType.MESH,
            )
        pl.semaphore_wait(barrier_sem, 2)

        xb_ref[...] = x_ref[...].astype(jnp.bfloat16)

        def make(src_ref, dst_ref, off, partner, i):
            return pltpu.make_async_remote_copy(
                src_ref=src_ref.at[pl.ds(off, bs), :],
                dst_ref=dst_ref.at[pl.ds(off, bs), :],
                send_sem=send_sems.at[i],
                recv_sem=recv_sems.at[i],
                device_id=(partner,),
                device_id_type=pl.DeviceIdType.MESH,
            )

        def acc(off):
            r1_ref[pl.ds(off, bs), :] = (
                r1_ref[pl.ds(off, bs), :] + xb_ref[pl.ds(off, bs), :]
            )

        t = {}
        for i, (src, dst, off, tgt) in {
            0: (xb_ref, r1_ref, rowA(p2 ^ 1), p1),
            1: (xb_ref, r1_ref, rowA(p1), p1),
            2: (xb_ref, r1_ref, rowB(p2 ^ 1), p2),
            3: (xb_ref, r1_ref, rowB(p2), p2),
        }.items():
            t[i] = make(src, dst, off, tgt, i)
            t[i].start()

        t[0].wait_recv()
        acc(rowA(p2))
        t[4] = make(r1_ref, r2_ref, rowA(p2), p2, 4)
        t[4].start()

        t[2].wait_recv()
        acc(rowB(p1))
        t[5] = make(r1_ref, r2_ref, rowB(p1), p1, 5)
        t[5].start()

        t[1].wait_recv()
        acc(rowA(my))
        t[3].wait_recv()
        acc(rowB(my))

        def norm_block(off):
            s = r1_ref[pl.ds(off, bs), :] + r2_ref[pl.ds(off, bs), :]
            y = s.astype(jnp.float32) + resid_ref[pl.ds(off, bs), :]
            ms = jnp.mean(y * y, axis=-1, keepdims=True)
            o = y * lax.rsqrt(ms + 1e-6) * gamma_ref[...]
            out_ref[pl.ds(off, bs), :] = o
            ag_ref[pl.ds(off, bs), :] = o.astype(jnp.bfloat16)

        t[4].wait_recv()
        norm_block(rowA(my))
        t[6] = make(ag_ref, ag_ref, rowA(my), p2, 6)
        t[8] = make(ag_ref, ag_ref, rowA(my), p1, 8)
        t[6].start()
        t[8].start()

        t[5].wait_recv()
        norm_block(rowB(my))
        t[7] = make(ag_ref, ag_ref, rowB(my), p1, 7)
        t[10] = make(ag_ref, ag_ref, rowB(my), p2, 10)
        t[7].start()
        t[10].start()

        t[6].wait_recv()
        t[9] = make(ag_ref, ag_ref, rowA(p2), p1, 9)
        t[9].start()
        out_ref[pl.ds(rowA(p2), bs), :] = (
            ag_ref[pl.ds(rowA(p2), bs), :].astype(jnp.float32)
        )

        t[7].wait_recv()
        t[11] = make(ag_ref, ag_ref, rowB(p1), p2, 11)
        t[11].start()
        out_ref[pl.ds(rowB(p1), bs), :] = (
            ag_ref[pl.ds(rowB(p1), bs), :].astype(jnp.float32)
        )

        for i, off in ((8, rowA(p1)), (9, rowA(p2 ^ 1)),
                       (10, rowB(p2)), (11, rowB(p2 ^ 1))):
            t[i].wait_recv()
            out_ref[pl.ds(off, bs), :] = (
                ag_ref[pl.ds(off, bs), :].astype(jnp.float32)
            )

        for i in range(12):
            t[i].wait_send()

    return pl.pallas_call(
        body,
        out_shape=jax.ShapeDtypeStruct((m, n), jnp.float32),
        in_specs=[
            pl.BlockSpec(memory_space=pltpu.VMEM),
            pl.BlockSpec(memory_space=pltpu.VMEM),
            pl.BlockSpec(memory_space=pltpu.VMEM),
        ],
        out_specs=pl.BlockSpec(memory_space=pltpu.VMEM),
        scratch_shapes=[
            pltpu.VMEM((m, n), jnp.bfloat16),
            pltpu.VMEM((m, n), jnp.bfloat16),
            pltpu.VMEM((m, n), jnp.bfloat16),
            pltpu.VMEM((m, n), jnp.bfloat16),
            pltpu.SemaphoreType.DMA((12,)),
            pltpu.SemaphoreType.DMA((12,)),
        ],
        compiler_params=pltpu.CompilerParams(collective_id=0),
    )(x, resid, gamma2d)
